# Initial kernel scaffold; baseline (speedup 1.0000x reference)
import jax
import jax.numpy as jnp
from jax.experimental import pallas as pl


def kernel(neigh_idx, node_features, W_agg, b_agg, W_upd):
    raise NotImplementedError("write your pallas kernel here")


# speedup vs baseline: 10.1668x; 10.1668x over previous
import functools

import jax
import jax.numpy as jnp
from jax import lax
from jax.experimental import pallas as pl
from jax.experimental.pallas import tpu as pltpu
from jax.experimental.pallas import tpu_sc as plsc

_N = 100000
_S = 16
_D = 32
_DEPTH = 2

_NC = 2
_NS = 16
_NW = _NC * _NS
_NODES_PER_TILE = _N // _NW
_P = 5
_G = _P * _S
_CH = _NODES_PER_TILE // _P
_NBUF = 5

_LANES = 16


def _gather_max_sc(f, idx3):
    mesh = plsc.VectorSubcoreMesh(core_axis_name="c", subcore_axis_name="s")

    @functools.partial(
        pl.kernel,
        mesh=mesh,
        compiler_params=pltpu.CompilerParams(use_tc_tiling_on_sc=False),
        out_type=jax.ShapeDtypeStruct((_N * _D,), jnp.float32),
        scratch_types=[
            pltpu.VMEM((_CH, _G), jnp.int32),
            pltpu.VMEM((_NBUF, _G, _D), jnp.float32),
            pltpu.VMEM((_NBUF, _P * _D), jnp.float32),
        ] + [pltpu.SemaphoreType.DMA] * (2 * _NBUF),
    )
    def k(f_hbm, idx_hbm, out_hbm, idx_v, rows_v, out_v, *sems):
        gsem = sems[:_NBUF]
        osem = sems[_NBUF:]
        wid = lax.axis_index("s") * _NC + lax.axis_index("c")
        node_base = wid * _NODES_PER_TILE

        pltpu.sync_copy(idx_hbm.at[wid], idx_v)

        def g_start(c, b):
            pltpu.async_copy(f_hbm.at[idx_v.at[c]], rows_v.at[b], gsem[b])

        def g_wait(c, b):
            pltpu.make_async_copy(
                f_hbm.at[idx_v.at[c]], rows_v.at[b], gsem[b]).wait()

        def o_start(c, b):
            pltpu.async_copy(
                out_v.at[b],
                out_hbm.at[pl.ds((node_base + c * _P) * _D, _P * _D)],
                osem[b])

        def o_wait(c, b):
            pltpu.make_async_copy(
                out_v.at[b],
                out_hbm.at[pl.ds((node_base + c * _P) * _D, _P * _D)],
                osem[b]).wait()

        for b in range(_NBUF):
            g_start(b, b)

        def body(i, carry):
            for b in range(_NBUF):
                c = i * _NBUF + b
                g_wait(c, b)

                @pl.when(i > 0)
                def _():
                    o_wait(c - _NBUF, b)

                for p in range(_P):
                    r0 = p * _S
                    a0 = rows_v[b, r0, 0:_LANES]
                    a1 = rows_v[b, r0, _LANES:_D]
                    for s in range(1, _S):
                        a0 = jnp.maximum(a0, rows_v[b, r0 + s, 0:_LANES])
                        a1 = jnp.maximum(a1, rows_v[b, r0 + s, _LANES:_D])
                    out_v[b, p * _D:p * _D + _LANES] = a0
                    out_v[b, p * _D + _LANES:(p + 1) * _D] = a1

                o_start(c, b)

                @pl.when(i < _CH // _NBUF - 1)
                def _():
                    g_start(c + _NBUF, b)
            return carry

        lax.fori_loop(0, _CH // _NBUF, body, 0)

        for b in range(_NBUF):
            o_wait(_CH - _NBUF + b, b)

    return k(f, idx3).reshape(_N, _D)


_BLK = 4000


def _tc_f(emb, wat, b2):
    def body(e_ref, w_ref, b_ref, f_ref):
        f_ref[...] = jnp.maximum(
            jnp.dot(e_ref[...], w_ref[...],
                    preferred_element_type=jnp.float32) + b_ref[...], 0.0)

    return pl.pallas_call(
        body,
        grid=(_N // _BLK,),
        in_specs=[
            pl.BlockSpec((_BLK, _D), lambda i: (i, 0)),
            pl.BlockSpec((_D, _D), lambda i: (0, 0)),
            pl.BlockSpec((1, _D), lambda i: (0, 0)),
        ],
        out_specs=pl.BlockSpec((_BLK, _D), lambda i: (i, 0)),
        out_shape=jax.ShapeDtypeStruct((_N, _D), jnp.float32),
    )(emb, wat, b2)


def _tc_update(emb, pooled, w1t, w2t, wat, b2):
    def body(e_ref, p_ref, w1_ref, w2_ref, wa_ref, b_ref, eo_ref, fo_ref):
        u = jnp.dot(e_ref[...], w1_ref[...],
                    preferred_element_type=jnp.float32)
        u = u + jnp.dot(p_ref[...], w2_ref[...],
                        preferred_element_type=jnp.float32)
        u = jnp.maximum(u, 0.0)
        nrm = jnp.sqrt(jnp.sum(u * u, axis=1, keepdims=True))
        e = u / jnp.maximum(nrm, 1e-12)
        eo_ref[...] = e
        fo_ref[...] = jnp.maximum(
            jnp.dot(e, wa_ref[...],
                    preferred_element_type=jnp.float32) + b_ref[...], 0.0)

    return pl.pallas_call(
        body,
        grid=(_N // _BLK,),
        in_specs=[
            pl.BlockSpec((_BLK, _D), lambda i: (i, 0)),
            pl.BlockSpec((_BLK, _D), lambda i: (i, 0)),
            pl.BlockSpec((_D, _D), lambda i: (0, 0)),
            pl.BlockSpec((_D, _D), lambda i: (0, 0)),
            pl.BlockSpec((_D, _D), lambda i: (0, 0)),
            pl.BlockSpec((1, _D), lambda i: (0, 0)),
        ],
        out_specs=[
            pl.BlockSpec((_BLK, _D), lambda i: (i, 0)),
            pl.BlockSpec((_BLK, _D), lambda i: (i, 0)),
        ],
        out_shape=[
            jax.ShapeDtypeStruct((_N, _D), jnp.float32),
            jax.ShapeDtypeStruct((_N, _D), jnp.float32),
        ],
    )(emb, pooled, w1t, w2t, wat, b2)


def kernel(neigh_idx, node_features, W_agg, b_agg, W_upd):
    idx = neigh_idx.astype(jnp.int32).reshape(_DEPTH, _NW, _CH, _G)
    wat = W_agg.T
    b2 = b_agg.reshape(1, _D)
    w1t = W_upd[:, :_D].T
    w2t = W_upd[:, _D:].T

    emb = node_features
    f = _tc_f(emb, wat, b2)
    for k in range(_DEPTH):
        pooled = _gather_max_sc(f, idx[k])
        emb, f = _tc_update(emb, pooled, w1t, w2t, wat, b2)
    return emb
